# Initial kernel scaffold; baseline (speedup 1.0000x reference)
import jax
import jax.numpy as jnp
from jax.experimental import pallas as pl


def kernel(src, src_id, tgt, tgt_id, var_emb, enc, dec, out_w1, out_b1, out_g, out_b, out_w2, out_b2):
    raise NotImplementedError("write your pallas kernel here")


# speedup vs baseline: 1.0080x; 1.0080x over previous
import math
import functools

import numpy as np
import jax
import jax.numpy as jnp
from jax.experimental import pallas as pl
from jax.experimental.pallas import tpu as pltpu

B = 2
N_SRC = 2048
N_TGT = 2048
D = 768
H = 12
DH = D // H
DEPTH = 3
DFF = 4 * D
OUT_DIM = 768
N_VARS = 50
K_IND = 40
BUCKET = 64
N_HASHES = 4
N_SPECIAL = 4
MAX_VAR_LEN = 300

BR = 256
AG = 16


def _ln_kern(x_ref, g_ref, o_ref):
    x = x_ref[...]
    mu = jnp.mean(x, axis=-1, keepdims=True)
    var = jnp.mean((x - mu) ** 2, axis=-1, keepdims=True)
    o_ref[...] = (x - mu) * jax.lax.rsqrt(var + 1e-5) * g_ref[...]


def _pl_ln(x, g, interpret=False):
    b, n, d = x.shape
    xf = x.reshape(b * n, d)
    out = pl.pallas_call(
        _ln_kern,
        grid=((b * n) // BR,),
        in_specs=[
            pl.BlockSpec((BR, d), lambda i: (i, 0)),
            pl.BlockSpec((1, d), lambda i: (0, 0)),
        ],
        out_specs=pl.BlockSpec((BR, d), lambda i: (i, 0)),
        out_shape=jax.ShapeDtypeStruct((b * n, d), jnp.float32),
        interpret=interpret,
    )(xf, g.reshape(1, d))
    return out.reshape(b, n, d)


def _mm2_kern(x_ref, wa_ref, wb_ref, a_ref, b_ref):
    x = x_ref[...]
    a_ref[...] = jnp.dot(x, wa_ref[...], preferred_element_type=jnp.float32)
    b_ref[...] = jnp.dot(x, wb_ref[...], preferred_element_type=jnp.float32)


def _pl_mm2(x, wa, wb, interpret=False):
    b, n, d = x.shape
    xf = x.reshape(b * n, d)
    outs = pl.pallas_call(
        _mm2_kern,
        grid=((b * n) // BR,),
        in_specs=[
            pl.BlockSpec((BR, d), lambda i: (i, 0)),
            pl.BlockSpec((d, d), lambda i: (0, 0)),
            pl.BlockSpec((d, d), lambda i: (0, 0)),
        ],
        out_specs=[
            pl.BlockSpec((BR, d), lambda i: (i, 0)),
            pl.BlockSpec((BR, d), lambda i: (i, 0)),
        ],
        out_shape=[
            jax.ShapeDtypeStruct((b * n, d), jnp.float32),
            jax.ShapeDtypeStruct((b * n, d), jnp.float32),
        ],
        interpret=interpret,
    )(xf, wa, wb)
    return outs[0].reshape(b, n, d), outs[1].reshape(b, n, d)


def _mm_res_kern(x_ref, o_ref, w_ref, y_ref):
    y_ref[...] = x_ref[...] + jnp.dot(
        o_ref[...], w_ref[...], preferred_element_type=jnp.float32)


def _pl_mm_res(x, o, w, interpret=False):
    b, n, d = x.shape
    out = pl.pallas_call(
        _mm_res_kern,
        grid=((b * n) // BR,),
        in_specs=[
            pl.BlockSpec((BR, d), lambda i: (i, 0)),
            pl.BlockSpec((BR, d), lambda i: (i, 0)),
            pl.BlockSpec((d, d), lambda i: (0, 0)),
        ],
        out_specs=pl.BlockSpec((BR, d), lambda i: (i, 0)),
        out_shape=jax.ShapeDtypeStruct((b * n, d), jnp.float32),
        interpret=interpret,
    )(x.reshape(b * n, d), o.reshape(b * n, d), w)
    return out.reshape(b, n, d)


def _ff_kern(x_ref, g_ref, w1a_ref, w1b_ref, w2_ref, o_ref):
    j = pl.program_id(1)
    x = x_ref[...]
    mu = jnp.mean(x, axis=-1, keepdims=True)
    var = jnp.mean((x - mu) ** 2, axis=-1, keepdims=True)
    h = (x - mu) * jax.lax.rsqrt(var + 1e-5) * g_ref[...]
    a = jnp.dot(h, w1a_ref[...], preferred_element_type=jnp.float32)
    bb = jnp.dot(h, w1b_ref[...], preferred_element_type=jnp.float32)
    c = jnp.dot(a * jax.nn.sigmoid(bb), w2_ref[...],
                preferred_element_type=jnp.float32)

    @pl.when(j == 0)
    def _():
        o_ref[...] = x + c

    @pl.when(j > 0)
    def _():
        o_ref[...] += c


def _pl_ff(x, g, w1, w2, interpret=False):
    b, n, d = x.shape
    dff = w2.shape[0]
    nj = dff // d
    out = pl.pallas_call(
        _ff_kern,
        grid=((b * n) // BR, nj),
        in_specs=[
            pl.BlockSpec((BR, d), lambda i, j: (i, 0)),
            pl.BlockSpec((1, d), lambda i, j: (0, 0)),
            pl.BlockSpec((d, d), lambda i, j: (0, j)),
            pl.BlockSpec((d, d), lambda i, j: (0, nj + j)),
            pl.BlockSpec((d, d), lambda i, j: (j, 0)),
        ],
        out_specs=pl.BlockSpec((BR, d), lambda i, j: (i, 0)),
        out_shape=jax.ShapeDtypeStruct((b * n, d), jnp.float32),
        interpret=interpret,
    )(x.reshape(b * n, d), g.reshape(1, d), w1, w1, w2)
    return out.reshape(b, n, d)


def _head_kern(y_ref, w1_ref, b1_ref, g_ref, bt_ref, w2_ref, b2_ref, o_ref):
    h = jnp.dot(y_ref[...], w1_ref[...],
                preferred_element_type=jnp.float32) + b1_ref[...]
    mu = jnp.mean(h, axis=-1, keepdims=True)
    var = jnp.mean((h - mu) ** 2, axis=-1, keepdims=True)
    h = (h - mu) * jax.lax.rsqrt(var + 1e-5) * g_ref[...] + bt_ref[...]
    h = jnp.maximum(h, 0.0)
    o_ref[...] = jnp.dot(h, w2_ref[...],
                         preferred_element_type=jnp.float32) + b2_ref[...]


def _pl_head(y, w1, b1, g, bt, w2, b2, interpret=False):
    b, n, d = y.shape
    od = w1.shape[1]
    out = pl.pallas_call(
        _head_kern,
        grid=((b * n) // BR,),
        in_specs=[
            pl.BlockSpec((BR, d), lambda i: (i, 0)),
            pl.BlockSpec((d, od), lambda i: (0, 0)),
            pl.BlockSpec((1, od), lambda i: (0, 0)),
            pl.BlockSpec((1, od), lambda i: (0, 0)),
            pl.BlockSpec((1, od), lambda i: (0, 0)),
            pl.BlockSpec((od, od), lambda i: (0, 0)),
            pl.BlockSpec((1, od), lambda i: (0, 0)),
        ],
        out_specs=pl.BlockSpec((BR, od), lambda i: (i, 0)),
        out_shape=jax.ShapeDtypeStruct((b * n, od), jnp.float32),
        interpret=interpret,
    )(y.reshape(b * n, d), w1, b1.reshape(1, od), g.reshape(1, od),
      bt.reshape(1, od), w2, b2.reshape(1, od))
    return out.reshape(b, n, od)


def _attn_kern(q_ref, qlb_ref, v_ref, vlb_ref, id_ref, idlb_ref,
               o_ref, lg_ref):
    q = q_ref[0]
    qlb = qlb_ref[0]
    v = v_ref[0]
    vlb = vlb_ref[0]
    ids = id_ref[0, :, :, 0]
    idlb = idlb_ref[0, :, :, 0]

    def _norm(z):
        return z / (jnp.sqrt(jnp.sum(z * z, axis=-1, keepdims=True)) + 1e-9)

    k_cur = _norm(q)
    k_prev = jnp.concatenate([_norm(qlb), k_cur[:-1]], axis=0)
    kk = jnp.concatenate([k_cur, k_prev], axis=1)
    vv = jnp.concatenate([v, jnp.concatenate([vlb, v[:-1]], axis=0)], axis=1)
    ids_k = jnp.concatenate([ids, jnp.concatenate([idlb, ids[:-1]], axis=0)],
                            axis=1)

    dots = jax.lax.dot_general(
        q, kk, (((2,), (2,)), ((0,), (0,))),
        preferred_element_type=jnp.float32) * (1.0 / 8.0)
    mask = ids[:, :, None] == ids_k[:, None, :]
    dots = jnp.where(mask, -5e4, dots)
    m = jnp.max(dots, axis=-1, keepdims=True)
    p = jnp.exp(dots - m)
    s = jnp.sum(p, axis=-1, keepdims=True)
    lse = m + jnp.log(s)
    o = jax.lax.dot_general(
        p / s, vv, (((2,), (1,)), ((0,), (0,))),
        preferred_element_type=jnp.float32)
    o_ref[0] = o
    lg_ref[0] = lse


def _pl_attn(sqk, sv, sids, interpret=False):
    bh, nc, c, dh = sqk.shape
    ng = nc // AG

    def lb(i, g):
        return (i, (g * AG - 1) % nc, 0, 0)

    outs = pl.pallas_call(
        _attn_kern,
        grid=(bh, ng),
        in_specs=[
            pl.BlockSpec((1, AG, c, dh), lambda i, g: (i, g, 0, 0)),
            pl.BlockSpec((1, 1, c, dh), lb),
            pl.BlockSpec((1, AG, c, dh), lambda i, g: (i, g, 0, 0)),
            pl.BlockSpec((1, 1, c, dh), lb),
            pl.BlockSpec((1, AG, c, 1), lambda i, g: (i, g, 0, 0)),
            pl.BlockSpec((1, 1, c, 1), lb),
        ],
        out_specs=[
            pl.BlockSpec((1, AG, c, dh), lambda i, g: (i, g, 0, 0)),
            pl.BlockSpec((1, AG, c, 1), lambda i, g: (i, g, 0, 0)),
        ],
        out_shape=[
            jax.ShapeDtypeStruct((bh, nc, c, dh), jnp.float32),
            jax.ShapeDtypeStruct((bh, nc, c, 1), jnp.float32),
        ],
        interpret=interpret,
    )(sqk, sqk, sv, sv, sids, sids)
    return outs[0], outs[1]


def _lsh_attention(qk, v, rng, interpret=False):
    b, h, n, dh = qk.shape
    n_buckets = n // BUCKET
    rot = jax.random.normal(rng, (dh, N_HASHES, n_buckets // 2), jnp.float32)
    rotated = jnp.einsum('bhnd,dkr->bhnkr', qk, rot)
    rotated = jnp.concatenate([rotated, -rotated], axis=-1)
    buckets = jnp.argmax(rotated, axis=-1)
    buckets = jnp.moveaxis(buckets, 3, 2).reshape(b, h, N_HASHES * n)
    offs = jnp.repeat(jnp.arange(N_HASHES), n) * n_buckets
    buckets = buckets + offs[None, None, :]
    m = N_HASHES * n
    ticker = jnp.arange(m) % n
    bt = buckets * n + ticker[None, None, :]
    sticker = jnp.argsort(bt, axis=-1)
    undo = jnp.argsort(sticker, axis=-1)
    st = sticker % n

    sqk = jnp.take_along_axis(qk, st[..., None], axis=2)
    sv = jnp.take_along_axis(v, st[..., None], axis=2)
    nc = m // BUCKET
    sqk = sqk.reshape(b * h, nc, BUCKET, dh)
    sv = sv.reshape(b * h, nc, BUCKET, dh)
    sids = st.astype(jnp.float32).reshape(b * h, nc, BUCKET, 1)

    so, slg = _pl_attn(sqk, sv, sids, interpret=interpret)

    o = so.reshape(b, h, m, dh)
    lg = slg.reshape(b, h, m)
    o = jnp.take_along_axis(o, undo[..., None], axis=2)
    lg = jnp.take_along_axis(lg, undo, axis=2)
    o = o.reshape(b, h, N_HASHES, n, dh)
    lg = lg.reshape(b, h, N_HASHES, n, 1)
    w = jnp.exp(lg - jax.nn.logsumexp(lg, axis=2, keepdims=True))
    return jnp.sum(o * w, axis=2)


def _split_heads(x):
    b, n, _ = x.shape
    return x.reshape(b, n, H, DH).transpose(0, 2, 1, 3)


def _merge_heads(x):
    b, h, n, dh = x.shape
    return x.transpose(0, 2, 1, 3).reshape(b, n, h * dh)


def _attn_layer(x, p, i, rng, keys=None, interpret=False):
    h = _pl_ln(x, p['g1'][i], interpret=interpret)
    ctx = h if keys is None else jnp.concatenate([h, keys], axis=1)
    qk, v = _pl_mm2(ctx, p['wqk'][i], p['wv'][i], interpret=interpret)
    o = _lsh_attention(_split_heads(qk), _split_heads(v), rng,
                       interpret=interpret)
    o = _merge_heads(o)[:, : x.shape[1]]
    return _pl_mm_res(x, o, p['wo'][i], interpret=interpret)


def _var_seq_const(total_len):
    seq = np.tile(np.arange(N_VARS, dtype=np.int32) + N_SPECIAL, K_IND)
    seq = np.concatenate(
        [seq, np.zeros((total_len - seq.shape[0],), np.int32)])
    return jnp.asarray(seq)


@functools.partial(jax.jit, static_argnames=('interpret',))
def _run(src, tgt, var_emb, enc, dec,
         out_w1, out_b1, out_g, out_b, out_w2, out_b2, interpret=False):
    scale = math.sqrt(D)
    src_seq = _var_seq_const(N_SRC)
    tgt_seq = _var_seq_const(N_TGT)
    x = (src + var_emb[src_seq][None]) * scale
    y = (tgt + var_emb[tgt_seq][None]) * scale
    base = jax.random.key(42)
    for i in range(DEPTH):
        x = _attn_layer(x, enc, i, jax.random.fold_in(base, i),
                        interpret=interpret)
        x = _pl_ff(x, enc['g2'][i], enc['w1'][i], enc['w2'][i],
                   interpret=interpret)
    mem = x
    for i in range(DEPTH):
        y = _attn_layer(y, dec, i, jax.random.fold_in(base, 100 + i),
                        keys=mem, interpret=interpret)
        y = _pl_ff(y, dec['g2'][i], dec['w1'][i], dec['w2'][i],
                   interpret=interpret)
    return _pl_head(y, out_w1, out_b1, out_g, out_b, out_w2, out_b2,
                    interpret=interpret)


def kernel(src, src_id, tgt, tgt_id, var_emb, enc, dec,
           out_w1, out_b1, out_g, out_b, out_w2, out_b2):
    return _run(src, tgt, var_emb, enc, dec,
                out_w1, out_b1, out_g, out_b, out_w2, out_b2)
